# baseline (speedup 1.0000x reference)
import jax
import jax.numpy as jnp
from jax import lax
from jax.experimental import pallas as pl
from jax.experimental.pallas import tpu as pltpu

H = 376
W = 376
HW = H * W
B = 4
G = 16
NG = HW // G
BR = 192
NB = (NG + BR - 1) // BR
CODE = 8
QUAD = 4
LOC_WEIGHT = 2.0
FOCAL_ALPHA = 0.25
FOCAL_GAMMA = 2.0


def _softplus_neg_abs(x):
    return jnp.log(1.0 + jnp.exp(-jnp.abs(x)))


def _loss_kernel(t_ref, cls_ref, bp_ref, hbl_ref, sp_ref, ql_ref, out_ref):
    s = pl.program_id(0)

    @pl.when(s == 0)
    def _init():
        for i in range(6):
            out_ref[i] = 0.0

    t = t_ref[...]
    gid = lax.broadcasted_iota(jnp.int32, (BR, G), 0) + s * BR
    valid = gid < NG
    pos = (t > 0.0) & valid
    neg = (t == 0.0) & valid
    m = pos | neg

    posf = pos.astype(jnp.float32)
    m_cnt = jnp.sum(m.astype(jnp.float32))
    n_pos = jnp.sum(posf)

    x = cls_ref[...]
    tb = t[None, :, :]
    z = jnp.exp(-jnp.abs(x))
    p = jnp.where(x >= 0.0, 1.0 / (1.0 + z), z / (1.0 + z))
    ce = jnp.maximum(x, 0.0) - x * tb + jnp.log(1.0 + z)
    p_t = p * tb + (1.0 - p) * (1.0 - tb)
    alpha_t = FOCAL_ALPHA * tb + (1.0 - FOCAL_ALPHA) * (1.0 - tb)
    om = 1.0 - p_t
    focal = alpha_t * om * om * ce
    s_focal = jnp.sum(jnp.where(m[None, :, :], focal, 0.0))

    lane8 = lax.broadcasted_iota(jnp.int32, (G, G * CODE), 1) // CODE
    row8 = lax.broadcasted_iota(jnp.int32, (G, G * CODE), 0)
    e8 = (lane8 == row8).astype(jnp.float32)
    lane4 = lax.broadcasted_iota(jnp.int32, (G, G * QUAD), 1) // QUAD
    row4 = lax.broadcasted_iota(jnp.int32, (G, G * QUAD), 0)
    e4 = (lane4 == row4).astype(jnp.float32)
    ex8 = lax.dot(posf, e8, preferred_element_type=jnp.float32)
    ex4 = lax.dot(posf, e4, preferred_element_type=jnp.float32)

    hbl = hbl_ref[...]
    hbls = hbl[0] + hbl[1] + hbl[2] + hbl[3]
    diff = bp_ref[...] - hbls
    ad = jnp.abs(diff)
    sl1 = jnp.where(ad < 1.0, 0.5 * diff * diff, ad - 0.5)
    s_sl1 = jnp.sum(jnp.where(ex8 > 0.0, sl1, 0.0))

    ql = ql_ref[...]
    qls = ql[0] + ql[1] + ql[2] + ql[3]
    spv = sp_ref[...]
    bce = jnp.maximum(spv, 0.0) - spv * qls + _softplus_neg_abs(spv)
    s_bce = jnp.sum(jnp.where(ex4 > 0.0, bce, 0.0))

    out_ref[0] += s_focal
    out_ref[1] += m_cnt
    out_ref[2] += n_pos
    out_ref[3] += s_sl1
    out_ref[4] += s_bce

    @pl.when(s == NB - 1)
    def _finish():
        sf = out_ref[0]
        mc = out_ref[1]
        np_ = out_ref[2]
        ssl = out_ref[3]
        sbc = out_ref[4]
        cls_loss = sf / jnp.maximum(mc, 1.0)
        reg_loss = ssl / jnp.maximum(np_, 1.0) * LOC_WEIGHT
        spa_loss = sbc / jnp.maximum(np_ * QUAD, 1.0)
        out_ref[5] = cls_loss + reg_loss + spa_loss


def kernel(cls_preds, box_preds, spa_preds, heatmaps, hos_box_labels, quadrant_labels):
    t2 = heatmaps[0, 0].reshape(NG, G)
    cls3 = cls_preds.reshape(B, NG, G)
    bp2 = box_preds.reshape(NG, G * CODE)
    hbl3 = hos_box_labels.reshape(B, NG, G * CODE)
    sp2 = spa_preds.reshape(NG, G * QUAD)
    ql3 = quadrant_labels.reshape(B, NG, G * QUAD)

    out = pl.pallas_call(
        _loss_kernel,
        grid=(NB,),
        in_specs=[
            pl.BlockSpec((BR, G), lambda s: (s, 0)),
            pl.BlockSpec((B, BR, G), lambda s: (0, s, 0)),
            pl.BlockSpec((BR, G * CODE), lambda s: (s, 0)),
            pl.BlockSpec((B, BR, G * CODE), lambda s: (0, s, 0)),
            pl.BlockSpec((BR, G * QUAD), lambda s: (s, 0)),
            pl.BlockSpec((B, BR, G * QUAD), lambda s: (0, s, 0)),
        ],
        out_specs=pl.BlockSpec(memory_space=pltpu.SMEM),
        out_shape=jax.ShapeDtypeStruct((6,), jnp.float32),
    )(t2, cls3, bp2, hbl3, sp2, ql3)
    return out[5]


# speedup vs baseline: 1.2544x; 1.2544x over previous
import numpy as np
import jax
import jax.numpy as jnp
from jax import lax
from jax.experimental import pallas as pl
from jax.experimental.pallas import tpu as pltpu

H = 376
W = 376
HW = H * W
B = 4
BR = 8
NB = H // BR
CODE = 8
QUAD = 4
LOC_WEIGHT = 2.0
FOCAL_ALPHA = 0.25

_E8 = np.equal.outer(np.arange(W), np.arange(W * CODE) // CODE).astype(np.float32)
_E4 = np.equal.outer(np.arange(W), np.arange(W * QUAD) // QUAD).astype(np.float32)


def _loss_kernel(t_ref, cls_ref, bp_ref, hbl_ref, sp_ref, ql_ref, e8_ref, e4_ref,
                 out_ref):
    s = pl.program_id(0)

    @pl.when(s == 0)
    def _init():
        for i in range(6):
            out_ref[i] = 0.0

    t = t_ref[...]
    pos = t > 0.0
    neg = t == 0.0
    m = pos | neg

    posf = pos.astype(jnp.float32)
    m_cnt = jnp.sum(m.astype(jnp.float32))
    n_pos = jnp.sum(posf)

    x = cls_ref[...]
    tb = t[None, :, :]
    z = jnp.exp(-jnp.abs(x))
    p = jnp.where(x >= 0.0, 1.0 / (1.0 + z), z / (1.0 + z))
    ce = jnp.maximum(x, 0.0) - x * tb + jnp.log(1.0 + z)
    p_t = p * tb + (1.0 - p) * (1.0 - tb)
    alpha_t = FOCAL_ALPHA * tb + (1.0 - FOCAL_ALPHA) * (1.0 - tb)
    om = 1.0 - p_t
    focal = alpha_t * om * om * ce
    s_focal = jnp.sum(jnp.where(m[None, :, :], focal, 0.0))

    ex8 = lax.dot(posf, e8_ref[...], preferred_element_type=jnp.float32)
    ex4 = lax.dot(posf, e4_ref[...], preferred_element_type=jnp.float32)

    hbl = hbl_ref[...]
    hbls = hbl[0] + hbl[1] + hbl[2] + hbl[3]
    diff = bp_ref[...] - hbls
    ad = jnp.abs(diff)
    sl1 = jnp.where(ad < 1.0, 0.5 * diff * diff, ad - 0.5)
    s_sl1 = jnp.sum(jnp.where(ex8 > 0.0, sl1, 0.0))

    ql = ql_ref[...]
    qls = ql[0] + ql[1] + ql[2] + ql[3]
    spv = sp_ref[...]
    bce = (jnp.maximum(spv, 0.0) - spv * qls
           + jnp.log(1.0 + jnp.exp(-jnp.abs(spv))))
    s_bce = jnp.sum(jnp.where(ex4 > 0.0, bce, 0.0))

    out_ref[0] += s_focal
    out_ref[1] += m_cnt
    out_ref[2] += n_pos
    out_ref[3] += s_sl1
    out_ref[4] += s_bce

    @pl.when(s == NB - 1)
    def _finish():
        cls_loss = out_ref[0] / jnp.maximum(out_ref[1], 1.0)
        reg_loss = out_ref[3] / jnp.maximum(out_ref[2], 1.0) * LOC_WEIGHT
        spa_loss = out_ref[4] / jnp.maximum(out_ref[2] * QUAD, 1.0)
        out_ref[5] = cls_loss + reg_loss + spa_loss


def kernel(cls_preds, box_preds, spa_preds, heatmaps, hos_box_labels, quadrant_labels):
    t2 = heatmaps[0, 0]
    cls3 = cls_preds.reshape(B, H, W)
    bp2 = box_preds.reshape(H, W * CODE)
    hbl3 = hos_box_labels.reshape(B, H, W * CODE)
    sp2 = spa_preds.reshape(H, W * QUAD)
    ql3 = quadrant_labels.reshape(B, H, W * QUAD)

    out = pl.pallas_call(
        _loss_kernel,
        grid=(NB,),
        in_specs=[
            pl.BlockSpec((BR, W), lambda s: (s, 0)),
            pl.BlockSpec((B, BR, W), lambda s: (0, s, 0)),
            pl.BlockSpec((BR, W * CODE), lambda s: (s, 0)),
            pl.BlockSpec((B, BR, W * CODE), lambda s: (0, s, 0)),
            pl.BlockSpec((BR, W * QUAD), lambda s: (s, 0)),
            pl.BlockSpec((B, BR, W * QUAD), lambda s: (0, s, 0)),
            pl.BlockSpec((W, W * CODE), lambda s: (0, 0)),
            pl.BlockSpec((W, W * QUAD), lambda s: (0, 0)),
        ],
        out_specs=pl.BlockSpec(memory_space=pltpu.SMEM),
        out_shape=jax.ShapeDtypeStruct((6,), jnp.float32),
    )(t2, cls3, bp2, hbl3, sp2, ql3, jnp.asarray(_E8), jnp.asarray(_E4))
    return out[5]


# speedup vs baseline: 26.9457x; 21.4809x over previous
import jax
import jax.numpy as jnp
from jax import lax
from jax.experimental import pallas as pl
from jax.experimental.pallas import tpu as pltpu

H = 376
W = 376
HW = H * W
B = 4
BR = 8
PB = 3072
NB = H // BR
CODE = 8
QUAD = 4
LOC_WEIGHT = 2.0
FOCAL_ALPHA = 0.25


def _loss_kernel(t_ref, cls_ref, tf_ref, bp_ref, hbl_ref, sp_ref, ql_ref, out_ref):
    s = pl.program_id(0)

    @pl.when(s == 0)
    def _init():
        for i in range(6):
            out_ref[i] = 0.0

    t = t_ref[...]
    pos = t > 0.0
    m = pos | (t == 0.0)

    m_cnt = jnp.sum(m.astype(jnp.float32))
    n_pos = jnp.sum(pos.astype(jnp.float32))

    x = cls_ref[...]
    tb = t[None, :, :]
    z = jnp.exp(-jnp.abs(x))
    p = jnp.where(x >= 0.0, 1.0 / (1.0 + z), z / (1.0 + z))
    ce = jnp.maximum(x, 0.0) - x * tb + jnp.log(1.0 + z)
    p_t = p * tb + (1.0 - p) * (1.0 - tb)
    alpha_t = FOCAL_ALPHA * tb + (1.0 - FOCAL_ALPHA) * (1.0 - tb)
    om = 1.0 - p_t
    focal = alpha_t * om * om * ce
    s_focal = jnp.sum(jnp.where(m[None, :, :], focal, 0.0))

    tf = tf_ref[...]
    inb = (lax.iota(jnp.int32, PB) + s * PB) < HW
    mflat = ((tf > 0.0) & inb)[None, :]

    hbl = hbl_ref[...]
    hbls = hbl[0] + hbl[1] + hbl[2] + hbl[3]
    diff = bp_ref[...] - hbls
    ad = jnp.abs(diff)
    sl1 = jnp.where(ad < 1.0, 0.5 * diff * diff, ad - 0.5)
    s_sl1 = jnp.sum(jnp.where(mflat, sl1, 0.0))

    ql = ql_ref[...]
    qls = ql[0] + ql[1] + ql[2] + ql[3]
    spv = sp_ref[...]
    bce = (jnp.maximum(spv, 0.0) - spv * qls
           + jnp.log(1.0 + jnp.exp(-jnp.abs(spv))))
    s_bce = jnp.sum(jnp.where(mflat, bce, 0.0))

    out_ref[0] += s_focal
    out_ref[1] += m_cnt
    out_ref[2] += n_pos
    out_ref[3] += s_sl1
    out_ref[4] += s_bce

    @pl.when(s == NB - 1)
    def _finish():
        cls_loss = out_ref[0] / jnp.maximum(out_ref[1], 1.0)
        reg_loss = out_ref[3] / jnp.maximum(out_ref[2], 1.0) * LOC_WEIGHT
        spa_loss = out_ref[4] / jnp.maximum(out_ref[2] * QUAD, 1.0)
        out_ref[5] = cls_loss + reg_loss + spa_loss


def kernel(cls_preds, box_preds, spa_preds, heatmaps, hos_box_labels, quadrant_labels):
    t2 = heatmaps[0, 0]
    tflat = t2.reshape(HW)
    cls3 = cls_preds.reshape(B, H, W)
    bpT = box_preds.T
    hblT = jnp.transpose(hos_box_labels, (0, 1, 3, 2)).reshape(B, CODE, HW)
    spT = spa_preds.T
    qlT = jnp.transpose(quadrant_labels, (0, 1, 3, 2)).reshape(B, QUAD, HW)

    out = pl.pallas_call(
        _loss_kernel,
        grid=(NB,),
        in_specs=[
            pl.BlockSpec((BR, W), lambda s: (s, 0)),
            pl.BlockSpec((B, BR, W), lambda s: (0, s, 0)),
            pl.BlockSpec((PB,), lambda s: (s,)),
            pl.BlockSpec((CODE, PB), lambda s: (0, s)),
            pl.BlockSpec((B, CODE, PB), lambda s: (0, 0, s)),
            pl.BlockSpec((QUAD, PB), lambda s: (0, s)),
            pl.BlockSpec((B, QUAD, PB), lambda s: (0, 0, s)),
        ],
        out_specs=pl.BlockSpec(memory_space=pltpu.SMEM),
        out_shape=jax.ShapeDtypeStruct((6,), jnp.float32),
    )(t2, cls3, tflat, bpT, hblT, spT, qlT)
    return out[5]


# speedup vs baseline: 36.4742x; 1.3536x over previous
import jax
import jax.numpy as jnp
from jax import lax
from jax.experimental import pallas as pl
from jax.experimental.pallas import tpu as pltpu

H = 376
W = 376
HW = H * W
B = 4
BR = 16
PB = 6144
NB = (H + BR - 1) // BR
CODE = 8
QUAD = 4
LOC_WEIGHT = 2.0
FOCAL_ALPHA = 0.25


def _loss_kernel(t_ref, cls_ref, tf_ref, bp_ref, hbl_ref, sp_ref, ql_ref, out_ref):
    s = pl.program_id(0)

    @pl.when(s == 0)
    def _init():
        for i in range(6):
            out_ref[i] = 0.0

    t = t_ref[...]
    rowok = (lax.broadcasted_iota(jnp.int32, (BR, W), 0) + s * BR) < H
    pos = (t > 0.0) & rowok
    m = pos | ((t == 0.0) & rowok)

    m_cnt = jnp.sum(m.astype(jnp.float32))
    n_pos = jnp.sum(pos.astype(jnp.float32))

    x = cls_ref[...]
    tb = t[None, :, :]
    z = jnp.exp(-jnp.abs(x))
    p = jnp.where(x >= 0.0, 1.0 / (1.0 + z), z / (1.0 + z))
    ce = jnp.maximum(x, 0.0) - x * tb + jnp.log(1.0 + z)
    p_t = p * tb + (1.0 - p) * (1.0 - tb)
    alpha_t = FOCAL_ALPHA * tb + (1.0 - FOCAL_ALPHA) * (1.0 - tb)
    om = 1.0 - p_t
    focal = alpha_t * om * om * ce
    s_focal = jnp.sum(jnp.where(m[None, :, :], focal, 0.0))

    tf = tf_ref[...]
    inb = (lax.iota(jnp.int32, PB) + s * PB) < HW
    mflat = ((tf > 0.0) & inb)[None, :]

    hbl = hbl_ref[...]
    hbls = hbl[0] + hbl[1] + hbl[2] + hbl[3]
    diff = bp_ref[...] - hbls
    ad = jnp.abs(diff)
    sl1 = jnp.where(ad < 1.0, 0.5 * diff * diff, ad - 0.5)
    s_sl1 = jnp.sum(jnp.where(mflat, sl1, 0.0))

    ql = ql_ref[...]
    qls = ql[0] + ql[1] + ql[2] + ql[3]
    spv = sp_ref[...]
    bce = (jnp.maximum(spv, 0.0) - spv * qls
           + jnp.log(1.0 + jnp.exp(-jnp.abs(spv))))
    s_bce = jnp.sum(jnp.where(mflat, bce, 0.0))

    out_ref[0] += s_focal
    out_ref[1] += m_cnt
    out_ref[2] += n_pos
    out_ref[3] += s_sl1
    out_ref[4] += s_bce

    @pl.when(s == NB - 1)
    def _finish():
        cls_loss = out_ref[0] / jnp.maximum(out_ref[1], 1.0)
        reg_loss = out_ref[3] / jnp.maximum(out_ref[2], 1.0) * LOC_WEIGHT
        spa_loss = out_ref[4] / jnp.maximum(out_ref[2] * QUAD, 1.0)
        out_ref[5] = cls_loss + reg_loss + spa_loss


def kernel(cls_preds, box_preds, spa_preds, heatmaps, hos_box_labels, quadrant_labels):
    t2 = heatmaps[0, 0]
    tflat = t2.reshape(HW)
    cls3 = cls_preds.reshape(B, H, W)
    bpT = box_preds.T
    hblT = jnp.transpose(hos_box_labels, (0, 1, 3, 2)).reshape(B, CODE, HW)
    spT = spa_preds.T
    qlT = jnp.transpose(quadrant_labels, (0, 1, 3, 2)).reshape(B, QUAD, HW)

    out = pl.pallas_call(
        _loss_kernel,
        grid=(NB,),
        in_specs=[
            pl.BlockSpec((BR, W), lambda s: (s, 0)),
            pl.BlockSpec((B, BR, W), lambda s: (0, s, 0)),
            pl.BlockSpec((PB,), lambda s: (s,)),
            pl.BlockSpec((CODE, PB), lambda s: (0, s)),
            pl.BlockSpec((B, CODE, PB), lambda s: (0, 0, s)),
            pl.BlockSpec((QUAD, PB), lambda s: (0, s)),
            pl.BlockSpec((B, QUAD, PB), lambda s: (0, 0, s)),
        ],
        out_specs=pl.BlockSpec(memory_space=pltpu.SMEM),
        out_shape=jax.ShapeDtypeStruct((6,), jnp.float32),
    )(t2, cls3, tflat, bpT, hblT, spT, qlT)
    return out[5]


# speedup vs baseline: 44.1200x; 1.2096x over previous
import jax
import jax.numpy as jnp
from jax import lax
from jax.experimental import pallas as pl
from jax.experimental.pallas import tpu as pltpu

H = 376
W = 376
HW = H * W
B = 4
BR = 32
PB = 12288
NB = (H + BR - 1) // BR
CODE = 8
QUAD = 4
LOC_WEIGHT = 2.0
FOCAL_ALPHA = 0.25


def _loss_kernel(t_ref, cls_ref, tf_ref, bp_ref, hbl_ref, sp_ref, ql_ref, out_ref):
    s = pl.program_id(0)

    @pl.when(s == 0)
    def _init():
        for i in range(6):
            out_ref[i] = 0.0

    t = t_ref[...]
    rowok = (lax.broadcasted_iota(jnp.int32, (BR, W), 0) + s * BR) < H
    pos = (t > 0.0) & rowok
    m = pos | ((t == 0.0) & rowok)

    m_cnt = jnp.sum(m.astype(jnp.float32))
    n_pos = jnp.sum(pos.astype(jnp.float32))

    x = cls_ref[...]
    tb = t[None, :, :]
    z = jnp.exp(-jnp.abs(x))
    p = jnp.where(x >= 0.0, 1.0 / (1.0 + z), z / (1.0 + z))
    ce = jnp.maximum(x, 0.0) - x * tb + jnp.log(1.0 + z)
    p_t = p * tb + (1.0 - p) * (1.0 - tb)
    alpha_t = FOCAL_ALPHA * tb + (1.0 - FOCAL_ALPHA) * (1.0 - tb)
    om = 1.0 - p_t
    focal = alpha_t * om * om * ce
    s_focal = jnp.sum(jnp.where(m[None, :, :], focal, 0.0))

    tf = tf_ref[...]
    inb = (lax.iota(jnp.int32, PB) + s * PB) < HW
    mflat = ((tf > 0.0) & inb)[None, :]

    hbl = hbl_ref[...]
    hbls = hbl[0] + hbl[1] + hbl[2] + hbl[3]
    diff = bp_ref[...] - hbls
    ad = jnp.abs(diff)
    sl1 = jnp.where(ad < 1.0, 0.5 * diff * diff, ad - 0.5)
    s_sl1 = jnp.sum(jnp.where(mflat, sl1, 0.0))

    ql = ql_ref[...]
    qls = ql[0] + ql[1] + ql[2] + ql[3]
    spv = sp_ref[...]
    bce = (jnp.maximum(spv, 0.0) - spv * qls
           + jnp.log(1.0 + jnp.exp(-jnp.abs(spv))))
    s_bce = jnp.sum(jnp.where(mflat, bce, 0.0))

    out_ref[0] += s_focal
    out_ref[1] += m_cnt
    out_ref[2] += n_pos
    out_ref[3] += s_sl1
    out_ref[4] += s_bce

    @pl.when(s == NB - 1)
    def _finish():
        cls_loss = out_ref[0] / jnp.maximum(out_ref[1], 1.0)
        reg_loss = out_ref[3] / jnp.maximum(out_ref[2], 1.0) * LOC_WEIGHT
        spa_loss = out_ref[4] / jnp.maximum(out_ref[2] * QUAD, 1.0)
        out_ref[5] = cls_loss + reg_loss + spa_loss


def kernel(cls_preds, box_preds, spa_preds, heatmaps, hos_box_labels, quadrant_labels):
    t2 = heatmaps[0, 0]
    tflat = t2.reshape(HW)
    cls3 = cls_preds.reshape(B, H, W)
    bpT = box_preds.T
    hblT = jnp.transpose(hos_box_labels, (0, 1, 3, 2)).reshape(B, CODE, HW)
    spT = spa_preds.T
    qlT = jnp.transpose(quadrant_labels, (0, 1, 3, 2)).reshape(B, QUAD, HW)

    out = pl.pallas_call(
        _loss_kernel,
        grid=(NB,),
        in_specs=[
            pl.BlockSpec((BR, W), lambda s: (s, 0)),
            pl.BlockSpec((B, BR, W), lambda s: (0, s, 0)),
            pl.BlockSpec((PB,), lambda s: (s,)),
            pl.BlockSpec((CODE, PB), lambda s: (0, s)),
            pl.BlockSpec((B, CODE, PB), lambda s: (0, 0, s)),
            pl.BlockSpec((QUAD, PB), lambda s: (0, s)),
            pl.BlockSpec((B, QUAD, PB), lambda s: (0, 0, s)),
        ],
        out_specs=pl.BlockSpec(memory_space=pltpu.SMEM),
        out_shape=jax.ShapeDtypeStruct((6,), jnp.float32),
    )(t2, cls3, tflat, bpT, hblT, spT, qlT)
    return out[5]


# speedup vs baseline: 47.9443x; 1.0867x over previous
import jax
import jax.numpy as jnp
from jax import lax
from jax.experimental import pallas as pl
from jax.experimental.pallas import tpu as pltpu

H = 376
W = 376
HW = H * W
B = 4
BR = 64
PB = 24576
NB = (H + BR - 1) // BR
CODE = 8
QUAD = 4
LOC_WEIGHT = 2.0
FOCAL_ALPHA = 0.25


def _loss_kernel(t_ref, cls_ref, tf_ref, bp_ref, hbl_ref, sp_ref, ql_ref, out_ref):
    s = pl.program_id(0)

    @pl.when(s == 0)
    def _init():
        for i in range(6):
            out_ref[i] = 0.0

    t = t_ref[...]
    rowok = (lax.broadcasted_iota(jnp.int32, (BR, W), 0) + s * BR) < H
    pos = (t > 0.0) & rowok
    m = pos | ((t == 0.0) & rowok)

    m_cnt = jnp.sum(m.astype(jnp.float32))
    n_pos = jnp.sum(pos.astype(jnp.float32))

    x = cls_ref[...]
    tb = t[None, :, :]
    z = jnp.exp(-jnp.abs(x))
    p = jnp.where(x >= 0.0, 1.0 / (1.0 + z), z / (1.0 + z))
    ce = jnp.maximum(x, 0.0) - x * tb + jnp.log(1.0 + z)
    p_t = p * tb + (1.0 - p) * (1.0 - tb)
    alpha_t = FOCAL_ALPHA * tb + (1.0 - FOCAL_ALPHA) * (1.0 - tb)
    om = 1.0 - p_t
    focal = alpha_t * om * om * ce
    s_focal = jnp.sum(jnp.where(m[None, :, :], focal, 0.0))

    tf = tf_ref[...]
    inb = (lax.iota(jnp.int32, PB) + s * PB) < HW
    mflat = ((tf > 0.0) & inb)[None, :]

    hbl = hbl_ref[...]
    hbls = hbl[0] + hbl[1] + hbl[2] + hbl[3]
    diff = bp_ref[...] - hbls
    ad = jnp.abs(diff)
    sl1 = jnp.where(ad < 1.0, 0.5 * diff * diff, ad - 0.5)
    s_sl1 = jnp.sum(jnp.where(mflat, sl1, 0.0))

    ql = ql_ref[...]
    qls = ql[0] + ql[1] + ql[2] + ql[3]
    spv = sp_ref[...]
    bce = (jnp.maximum(spv, 0.0) - spv * qls
           + jnp.log(1.0 + jnp.exp(-jnp.abs(spv))))
    s_bce = jnp.sum(jnp.where(mflat, bce, 0.0))

    out_ref[0] += s_focal
    out_ref[1] += m_cnt
    out_ref[2] += n_pos
    out_ref[3] += s_sl1
    out_ref[4] += s_bce

    @pl.when(s == NB - 1)
    def _finish():
        cls_loss = out_ref[0] / jnp.maximum(out_ref[1], 1.0)
        reg_loss = out_ref[3] / jnp.maximum(out_ref[2], 1.0) * LOC_WEIGHT
        spa_loss = out_ref[4] / jnp.maximum(out_ref[2] * QUAD, 1.0)
        out_ref[5] = cls_loss + reg_loss + spa_loss


def kernel(cls_preds, box_preds, spa_preds, heatmaps, hos_box_labels, quadrant_labels):
    t2 = heatmaps[0, 0]
    tflat = t2.reshape(HW)
    cls3 = cls_preds.reshape(B, H, W)
    bpT = box_preds.T
    hblT = jnp.transpose(hos_box_labels, (0, 1, 3, 2)).reshape(B, CODE, HW)
    spT = spa_preds.T
    qlT = jnp.transpose(quadrant_labels, (0, 1, 3, 2)).reshape(B, QUAD, HW)

    out = pl.pallas_call(
        _loss_kernel,
        grid=(NB,),
        in_specs=[
            pl.BlockSpec((BR, W), lambda s: (s, 0)),
            pl.BlockSpec((B, BR, W), lambda s: (0, s, 0)),
            pl.BlockSpec((PB,), lambda s: (s,)),
            pl.BlockSpec((CODE, PB), lambda s: (0, s)),
            pl.BlockSpec((B, CODE, PB), lambda s: (0, 0, s)),
            pl.BlockSpec((QUAD, PB), lambda s: (0, s)),
            pl.BlockSpec((B, QUAD, PB), lambda s: (0, 0, s)),
        ],
        out_specs=pl.BlockSpec(memory_space=pltpu.SMEM),
        out_shape=jax.ShapeDtypeStruct((6,), jnp.float32),
    )(t2, cls3, tflat, bpT, hblT, spT, qlT)
    return out[5]
